# baseline (speedup 1.0000x reference)
import jax
import jax.numpy as jnp
from jax import lax
from jax.experimental import pallas as pl
from jax.experimental.pallas import tpu as pltpu
from jax.experimental.pallas import tpu_sc as plsc

VOCAB = 1000000
ROWS = 65536
N_CHUNKS = 8
CHUNK = 16
B = 16384

NC = 2
NS = 16
NW = NC * NS
TPW = B // NW
RPW = TPW * N_CHUNKS
SUB = 4
CPS = N_CHUNKS // SUB
RPS = TPW * CPS
IDX_W = 128
KPW = RPW // IDX_W
K_SUB = RPS // IDX_W


def _body(x_hbm, t0_hbm, t1_hbm, w_hbm, h0_hbm, h1_hbm, h2_hbm, out_hbm,
          x_v, xe_v, g0_v, g1_v, g2_v, w0_v, w1_v, p0_v, p1_v, out_v, sem):
    c = lax.axis_index("c")
    s = lax.axis_index("s")
    wid = s * NC + c
    tok_base = wid * TPW

    pltpu.sync_copy(x_hbm.at[pl.ds(tok_base, TPW)], x_v)

    def xe_body(i, _):
        cc = lax.shift_right_logical(i, 5)
        bo = jnp.bitwise_and(i, 31) * 16
        xe_v[pl.ds(i * 16, 16)] = x_v[pl.ds(bo, 16)] + cc * VOCAB
        return 0

    lax.fori_loop(0, RPW // 16, xe_body, 0)

    cp0 = pltpu.async_copy(h0_hbm.at[xe_v], g0_v, sem)
    cp1 = pltpu.async_copy(h1_hbm.at[xe_v], g1_v, sem)
    cp2 = pltpu.async_copy(h2_hbm.at[xe_v], g2_v, sem)
    cp0.wait()
    cp1.wait()
    cp2.wait()

    def idx_body(i, _):
        cc = lax.shift_right_logical(i, 5)
        sl = pl.ds(i * 16, 16)
        g0_v[sl] = g0_v[sl] * N_CHUNKS + cc
        g1_v[sl] = g1_v[sl] * N_CHUNKS + cc
        w0i = g2_v[sl] + cc * (2 * ROWS)
        xe_v[sl] = w0i
        g2_v[sl] = w0i + ROWS
        return 0

    lax.fori_loop(0, RPW // 16, idx_body, 0)

    cpw0 = pltpu.async_copy(w_hbm.at[xe_v], w0_v, sem)
    cpw1 = pltpu.async_copy(w_hbm.at[g2_v], w1_v, sem)
    cpw0.wait()
    cpw1.wait()

    for sb in range(SUB):
        sl = pl.ds(sb * RPS, RPS)
        cpt0 = pltpu.async_copy(t0_hbm.at[g0_v.at[sl]], p0_v, sem)
        cpt1 = pltpu.async_copy(t1_hbm.at[g1_v.at[sl]], p1_v, sem)
        cpt0.wait()
        cpt1.wait()

        def row_body(m, _):
            q = sb * RPS + m
            qv = jnp.full((16,), q, jnp.int32)
            w0 = plsc.load_gather(w0_v, [qv])
            w1 = plsc.load_gather(w1_v, [qv])
            cc = sb * CPS + lax.shift_right_logical(m, 9)
            b = jnp.bitwise_and(m, TPW - 1)
            out_v[b, pl.ds(cc * CHUNK, CHUNK)] = (
                p0_v[m, :] * w0 + p1_v[m, :] * w1)
            return 0

        lax.fori_loop(0, RPS, row_body, 0)

    pltpu.sync_copy(out_v, out_hbm.at[pl.ds(tok_base, TPW), :])


@jax.jit
def _call(x, t0, t1, w, h0f, h1f, h2f):
    mesh = plsc.VectorSubcoreMesh(core_axis_name="c", subcore_axis_name="s")
    run = pl.kernel(
        _body,
        out_type=jax.ShapeDtypeStruct((B, N_CHUNKS * CHUNK), jnp.float32),
        mesh=mesh,
        compiler_params=pltpu.CompilerParams(use_tc_tiling_on_sc=False,
                                             needs_layout_passes=False),
        scratch_types=[
            pltpu.VMEM((TPW,), jnp.int32),
            pltpu.VMEM((RPW,), jnp.int32),
            pltpu.VMEM((RPW,), jnp.int32),
            pltpu.VMEM((RPW,), jnp.int32),
            pltpu.VMEM((RPW,), jnp.int32),
            pltpu.VMEM((RPW,), jnp.float32),
            pltpu.VMEM((RPW,), jnp.float32),
            pltpu.VMEM((RPS, CHUNK), jnp.float32),
            pltpu.VMEM((RPS, CHUNK), jnp.float32),
            pltpu.VMEM((TPW, 128), jnp.float32),
            pltpu.SemaphoreType.DMA,
        ],
    )
    return run(x, t0, t1, w, h0f, h1f, h2f)


def kernel(x, table0, table1, weights, h0, h1, h2):
    t0 = table0.reshape(ROWS * N_CHUNKS, CHUNK)
    t1 = table1.reshape(ROWS * N_CHUNKS, CHUNK)
    w = jnp.transpose(weights, (1, 2, 0)).reshape(N_CHUNKS * 2 * ROWS)
    h0f = jnp.transpose(h0).reshape(N_CHUNKS * VOCAB)
    h1f = jnp.transpose(h1).reshape(N_CHUNKS * VOCAB)
    h2f = jnp.transpose(h2).reshape(N_CHUNKS * VOCAB)
    return _call(x, t0, t1, w, h0f, h1f, h2f)


# speedup vs baseline: 1.2927x; 1.2927x over previous
import jax
import jax.numpy as jnp
from jax import lax
from jax.experimental import pallas as pl
from jax.experimental.pallas import tpu as pltpu
from jax.experimental.pallas import tpu_sc as plsc

VOCAB = 1000000
ROWS = 65536
N_CHUNKS = 8
CHUNK = 16
B = 16384

NC = 2
NS = 16
NW = NC * NS
TPW = B // NW
RPW = TPW * N_CHUNKS
RPS = TPW


def _body(x_hbm, t0_hbm, t1_hbm, w_hbm, h0_hbm, h1_hbm, h2_hbm, out_hbm,
          x_v, h0_v, h1_v, h2_v, i0_v, i1_v, iw0_v, iw1_v, w0_v, w1_v,
          p0_v, p1_v, out_v, sem):
    c = lax.axis_index("c")
    s = lax.axis_index("s")
    wid = s * NC + c
    tok_base = wid * TPW

    pltpu.sync_copy(x_hbm.at[pl.ds(tok_base, TPW)], x_v)

    cp0 = pltpu.async_copy(h0_hbm.at[x_v], h0_v, sem)
    cp1 = pltpu.async_copy(h1_hbm.at[x_v], h1_v, sem)
    cp2 = pltpu.async_copy(h2_hbm.at[x_v], h2_v, sem)
    cp0.wait()
    cp1.wait()
    cp2.wait()

    iota = lax.iota(jnp.int32, 16)

    def idx_body(i, _):
        cc = lax.shift_right_logical(i, 5)
        ccv = jnp.full((16,), cc, jnp.int32)
        bvec = jnp.bitwise_and(i, 31) * 16 + iota
        g0 = plsc.load_gather(h0_v, [bvec, ccv])
        g1 = plsc.load_gather(h1_v, [bvec, ccv])
        i2 = plsc.load_gather(h2_v, [bvec, ccv]) * N_CHUNKS + cc
        sl = pl.ds(i * 16, 16)
        i0_v[sl] = g0 * N_CHUNKS + cc
        i1_v[sl] = g1 * N_CHUNKS + cc
        iw0_v[sl] = i2 * 2
        iw1_v[sl] = i2 * 2 + 1
        return 0

    lax.fori_loop(0, RPW // 16, idx_body, 0)

    cpw0 = pltpu.async_copy(w_hbm.at[iw0_v], w0_v, sem)
    cpw1 = pltpu.async_copy(w_hbm.at[iw1_v], w1_v, sem)
    cpw0.wait()
    cpw1.wait()

    for sb in range(N_CHUNKS):
        sl = pl.ds(sb * RPS, RPS)
        cpt0 = pltpu.async_copy(t0_hbm.at[i0_v.at[sl]], p0_v, sem)
        cpt1 = pltpu.async_copy(t1_hbm.at[i1_v.at[sl]], p1_v, sem)
        cpt0.wait()
        cpt1.wait()

        def row_body(m, _):
            qv = jnp.full((16,), sb * RPS + m, jnp.int32)
            w0 = plsc.load_gather(w0_v, [qv])
            w1 = plsc.load_gather(w1_v, [qv])
            out_v[m, pl.ds(sb * CHUNK, CHUNK)] = (
                p0_v[m, :] * w0 + p1_v[m, :] * w1)
            return 0

        lax.fori_loop(0, RPS, row_body, 0)

    pltpu.sync_copy(out_v, out_hbm.at[pl.ds(tok_base, TPW), :])


@jax.jit
def _call(x, t0, t1, w, h0, h1, h2):
    mesh = plsc.VectorSubcoreMesh(core_axis_name="c", subcore_axis_name="s")
    run = pl.kernel(
        _body,
        out_type=jax.ShapeDtypeStruct((B, N_CHUNKS * CHUNK), jnp.float32),
        mesh=mesh,
        compiler_params=pltpu.CompilerParams(use_tc_tiling_on_sc=False,
                                             needs_layout_passes=False),
        scratch_types=[
            pltpu.VMEM((TPW,), jnp.int32),
            pltpu.VMEM((TPW, N_CHUNKS), jnp.int32),
            pltpu.VMEM((TPW, N_CHUNKS), jnp.int32),
            pltpu.VMEM((TPW, N_CHUNKS), jnp.int32),
            pltpu.VMEM((RPW,), jnp.int32),
            pltpu.VMEM((RPW,), jnp.int32),
            pltpu.VMEM((RPW,), jnp.int32),
            pltpu.VMEM((RPW,), jnp.int32),
            pltpu.VMEM((RPW,), jnp.float32),
            pltpu.VMEM((RPW,), jnp.float32),
            pltpu.VMEM((RPS, CHUNK), jnp.float32),
            pltpu.VMEM((RPS, CHUNK), jnp.float32),
            pltpu.VMEM((TPW, 128), jnp.float32),
            pltpu.SemaphoreType.DMA,
        ],
    )
    return run(x, t0, t1, w, h0, h1, h2)


def kernel(x, table0, table1, weights, h0, h1, h2):
    t0 = table0.reshape(ROWS * N_CHUNKS, CHUNK)
    t1 = table1.reshape(ROWS * N_CHUNKS, CHUNK)
    w = weights.reshape(ROWS * N_CHUNKS * 2)
    return _call(x, t0, t1, w, h0, h1, h2)


# speedup vs baseline: 2.8402x; 2.1970x over previous
import jax
import jax.numpy as jnp
from jax import lax
from jax.experimental import pallas as pl
from jax.experimental.pallas import tpu as pltpu
from jax.experimental.pallas import tpu_sc as plsc

VOCAB = 1000000
ROWS = 65536
N_CHUNKS = 8
CHUNK = 16
B = 16384

NC = 2
NS = 16
NW = NC * NS
TPW = B // NW
RPW = TPW * N_CHUNKS
SUB = 4
CPS = N_CHUNKS // SUB
RPS = TPW * CPS
IDX_W = 128
KPW = RPW // IDX_W
K_SUB = RPS // IDX_W


def _body(x_hbm, t0_hbm, t1_hbm, w_hbm, h0_hbm, h1_hbm, h2_hbm, out_hbm,
          x_v, xe_v, g0_v, g1_v, g2_v, w0_v, w1_v, p0_v, p1_v, out_v, sem):
    c = lax.axis_index("c")
    s = lax.axis_index("s")
    wid = s * NC + c
    tok_base = wid * TPW

    pltpu.sync_copy(x_hbm.at[pl.ds(tok_base, TPW)], x_v)

    def xe_body(i, _):
        cc = lax.shift_right_logical(i, 5)
        bo = jnp.bitwise_and(i, 31) * 16
        xv = x_v[pl.ds(bo, 16)]
        xe_v[pl.ds(i * 16, 16)] = (
            lax.shift_left(lax.shift_right_logical(xv, 7), 10)
            + cc * 128 + jnp.bitwise_and(xv, 127))
        return 0

    lax.fori_loop(0, RPW // 16, xe_body, 0)

    cp0 = pltpu.async_copy(h0_hbm.at[xe_v], g0_v, sem)
    cp1 = pltpu.async_copy(h1_hbm.at[xe_v], g1_v, sem)
    cp2 = pltpu.async_copy(h2_hbm.at[xe_v], g2_v, sem)
    cp0.wait()
    cp1.wait()
    cp2.wait()

    def idx_body(i, _):
        cc = lax.shift_right_logical(i, 5)
        sl = pl.ds(i * 16, 16)
        g0_v[sl] = g0_v[sl] * N_CHUNKS + cc
        g1_v[sl] = g1_v[sl] * N_CHUNKS + cc
        i2 = g2_v[sl] * N_CHUNKS + cc
        xe_v[sl] = i2 * 2
        g2_v[sl] = i2 * 2 + 1
        return 0

    lax.fori_loop(0, RPW // 16, idx_body, 0)

    cpw0 = pltpu.async_copy(w_hbm.at[xe_v], w0_v, sem)
    cpw1 = pltpu.async_copy(w_hbm.at[g2_v], w1_v, sem)
    cpw0.wait()
    cpw1.wait()

    for sb in range(SUB):
        sl = pl.ds(sb * RPS, RPS)
        cpt0 = pltpu.async_copy(t0_hbm.at[g0_v.at[sl]], p0_v, sem)
        cpt1 = pltpu.async_copy(t1_hbm.at[g1_v.at[sl]], p1_v, sem)
        cpt0.wait()
        cpt1.wait()

        def row_body(m, _):
            q = sb * RPS + m
            qv = jnp.full((16,), q, jnp.int32)
            w0 = plsc.load_gather(w0_v, [qv])
            w1 = plsc.load_gather(w1_v, [qv])
            cc = sb * CPS + lax.shift_right_logical(m, 9)
            b = jnp.bitwise_and(m, TPW - 1)
            out_v[b, pl.ds(cc * CHUNK, CHUNK)] = (
                p0_v[m, :] * w0 + p1_v[m, :] * w1)
            return 0

        lax.fori_loop(0, RPS, row_body, 0)

    pltpu.sync_copy(out_v, out_hbm.at[pl.ds(tok_base, TPW), :])


@jax.jit
def _call(x, t0, t1, w, h0f, h1f, h2f):
    mesh = plsc.VectorSubcoreMesh(core_axis_name="c", subcore_axis_name="s")
    run = pl.kernel(
        _body,
        out_type=jax.ShapeDtypeStruct((B, N_CHUNKS * CHUNK), jnp.float32),
        mesh=mesh,
        compiler_params=pltpu.CompilerParams(use_tc_tiling_on_sc=False,
                                             needs_layout_passes=False),
        scratch_types=[
            pltpu.VMEM((TPW,), jnp.int32),
            pltpu.VMEM((RPW,), jnp.int32),
            pltpu.VMEM((RPW,), jnp.int32),
            pltpu.VMEM((RPW,), jnp.int32),
            pltpu.VMEM((RPW,), jnp.int32),
            pltpu.VMEM((RPW,), jnp.float32),
            pltpu.VMEM((RPW,), jnp.float32),
            pltpu.VMEM((RPS, CHUNK), jnp.float32),
            pltpu.VMEM((RPS, CHUNK), jnp.float32),
            pltpu.VMEM((TPW, 128), jnp.float32),
            pltpu.SemaphoreType.DMA,
        ],
    )
    return run(x, t0, t1, w, h0f, h1f, h2f)


HB = 65536
TILES = (VOCAB + 127) // 128
PTILES = 7816
HPAD = PTILES * 128


def _lin_body(i0_ref, i1_ref, i2_ref, o0_ref, o1_ref, o2_ref):
    for i_ref, o_ref in ((i0_ref, o0_ref), (i1_ref, o1_ref),
                         (i2_ref, o2_ref)):
        o_ref[...] = jnp.transpose(
            i_ref[...].reshape(N_CHUNKS, HB // 128, 128), (1, 0, 2))


def _linearize(h0t, h1t, h2t):
    spec_i = pl.BlockSpec((N_CHUNKS, HB), lambda xb: (0, xb))
    spec_o = pl.BlockSpec((HB // 128, N_CHUNKS, 128), lambda xb: (xb, 0, 0))
    out_t = jax.ShapeDtypeStruct((PTILES, N_CHUNKS, 128), jnp.int32)
    return pl.pallas_call(
        _lin_body,
        grid=((VOCAB + HB - 1) // HB,),
        in_specs=[spec_i, spec_i, spec_i],
        out_specs=[spec_o, spec_o, spec_o],
        out_shape=[out_t, out_t, out_t],
    )(h0t, h1t, h2t)


def kernel(x, table0, table1, weights, h0, h1, h2):
    t0 = table0.reshape(ROWS * N_CHUNKS, CHUNK)
    t1 = table1.reshape(ROWS * N_CHUNKS, CHUNK)
    w = weights.reshape(ROWS * N_CHUNKS * 2)
    h0f, h1f, h2f = (h.reshape(PTILES * N_CHUNKS * 128)
                     for h in _linearize(jnp.transpose(h0), jnp.transpose(h1),
                                         jnp.transpose(h2)))
    return _call(x, t0, t1, w, h0f, h1f, h2f)


# speedup vs baseline: 4.5893x; 1.6158x over previous
import jax
import jax.numpy as jnp
from jax import lax
from jax.experimental import pallas as pl
from jax.experimental.pallas import tpu as pltpu
from jax.experimental.pallas import tpu_sc as plsc

VOCAB = 1000000
ROWS = 65536
N_CHUNKS = 8
CHUNK = 16
B = 16384

NC = 2
NS = 16
NW = NC * NS
TPW = B // NW
RPW = TPW * N_CHUNKS
SUB = 4
CPS = N_CHUNKS // SUB
RPS = TPW * CPS
IDX_W = 128
KPW = RPW // IDX_W
K_SUB = RPS // IDX_W


def _body(x_hbm, t0_hbm, t1_hbm, w_hbm, h0_hbm, h1_hbm, h2_hbm, out_hbm,
          x_v, xe_v, g0_v, g1_v, g2_v, w0_v, w1_v, p0_v, p1_v, out_v, sem):
    c = lax.axis_index("c")
    s = lax.axis_index("s")
    wid = s * NC + c
    tok_base = wid * TPW

    pltpu.sync_copy(x_hbm.at[pl.ds(tok_base, TPW)], x_v)

    def xe_body(i, _):
        cc = lax.shift_right_logical(i, 5)
        bo = jnp.bitwise_and(i, 31) * 16
        xv = x_v[pl.ds(bo, 16)]
        xe_v[pl.ds(i * 16, 16)] = (
            lax.shift_left(lax.shift_right_logical(xv, 7), 10)
            + cc * 128 + jnp.bitwise_and(xv, 127))
        return 0

    lax.fori_loop(0, RPW // 16, xe_body, 0)

    cp0 = pltpu.async_copy(h0_hbm.at[xe_v], g0_v, sem)
    cp1 = pltpu.async_copy(h1_hbm.at[xe_v], g1_v, sem)
    cp2 = pltpu.async_copy(h2_hbm.at[xe_v], g2_v, sem)
    cp0.wait()
    cp1.wait()
    cp2.wait()

    def idx_body(i, _):
        cc = lax.shift_right_logical(i, 5)
        sl = pl.ds(i * 16, 16)
        g0_v[sl] = g0_v[sl] * N_CHUNKS + cc
        g1_v[sl] = g1_v[sl] * N_CHUNKS + cc
        i2 = g2_v[sl] * N_CHUNKS + cc
        xe_v[sl] = i2 * 2
        g2_v[sl] = i2 * 2 + 1
        return 0

    lax.fori_loop(0, RPW // 16, idx_body, 0)

    cpw0 = pltpu.async_copy(w_hbm.at[xe_v], w0_v, sem)
    cpw1 = pltpu.async_copy(w_hbm.at[g2_v], w1_v, sem)
    cpw0.wait()
    cpw1.wait()

    for sb in range(SUB):
        sl = pl.ds(sb * RPS, RPS)
        cpt0 = pltpu.async_copy(t0_hbm.at[g0_v.at[sl]], p0_v, sem)
        cpt1 = pltpu.async_copy(t1_hbm.at[g1_v.at[sl]], p1_v, sem)
        cpt0.wait()
        cpt1.wait()

        def row_body(m, _):
            q = sb * RPS + m
            qv = jnp.full((16,), q, jnp.int32)
            w0 = plsc.load_gather(w0_v, [qv])
            w1 = plsc.load_gather(w1_v, [qv])
            cc = sb * CPS + lax.shift_right_logical(m, 9)
            b = jnp.bitwise_and(m, TPW - 1)
            out_v[b, pl.ds(cc * CHUNK, CHUNK)] = (
                p0_v[m, :] * w0 + p1_v[m, :] * w1)
            return 0

        lax.fori_loop(0, RPS, row_body, 0)

    pltpu.sync_copy(out_v, out_hbm.at[pl.ds(tok_base, TPW), :])


@jax.jit
def _call(x, t0, t1, w, h0f, h1f, h2f):
    mesh = plsc.VectorSubcoreMesh(core_axis_name="c", subcore_axis_name="s")
    run = pl.kernel(
        _body,
        out_type=jax.ShapeDtypeStruct((B, N_CHUNKS * CHUNK), jnp.float32),
        mesh=mesh,
        compiler_params=pltpu.CompilerParams(use_tc_tiling_on_sc=False,
                                             needs_layout_passes=False),
        scratch_types=[
            pltpu.VMEM((TPW,), jnp.int32),
            pltpu.VMEM((RPW,), jnp.int32),
            pltpu.VMEM((RPW,), jnp.int32),
            pltpu.VMEM((RPW,), jnp.int32),
            pltpu.VMEM((RPW,), jnp.int32),
            pltpu.VMEM((RPW,), jnp.float32),
            pltpu.VMEM((RPW,), jnp.float32),
            pltpu.VMEM((RPS, CHUNK), jnp.float32),
            pltpu.VMEM((RPS, CHUNK), jnp.float32),
            pltpu.VMEM((TPW, 128), jnp.float32),
            pltpu.SemaphoreType.DMA,
        ],
    )
    return run(x, t0, t1, w, h0f, h1f, h2f)


HB = 65536
TILES = (VOCAB + 127) // 128
PTILES = 7816
HPAD = PTILES * 128


def _lin_body(i0_ref, i1_ref, i2_ref, o0_ref, o1_ref, o2_ref):
    for i_ref, o_ref in ((i0_ref, o0_ref), (i1_ref, o1_ref),
                         (i2_ref, o2_ref)):
        o_ref[...] = jnp.transpose(
            i_ref[...].reshape(N_CHUNKS, HB // 128, 128), (1, 0, 2))


WT = 4096


def _tab_body(t0_ref, t1_ref, o0_ref, o1_ref):
    o0_ref[...] = jnp.transpose(t0_ref[...].reshape(128, WT))
    o1_ref[...] = jnp.transpose(t1_ref[...].reshape(128, WT))


def _tab_linearize(t0p, t1p):
    spec_i = pl.BlockSpec((N_CHUNKS, CHUNK, WT), lambda rb: (0, 0, rb))
    spec_o = pl.BlockSpec((WT, 128), lambda rb: (rb, 0))
    out_t = jax.ShapeDtypeStruct((ROWS, 128), jnp.float32)
    return pl.pallas_call(
        _tab_body,
        grid=(ROWS // WT,),
        in_specs=[spec_i, spec_i],
        out_specs=[spec_o, spec_o],
        out_shape=[out_t, out_t],
    )(t0p, t1p)


def _linearize(h0t, h1t, h2t):
    spec_i = pl.BlockSpec((N_CHUNKS, HB), lambda xb: (0, xb))
    spec_o = pl.BlockSpec((HB // 128, N_CHUNKS, 128), lambda xb: (xb, 0, 0))
    out_t = jax.ShapeDtypeStruct((PTILES, N_CHUNKS, 128), jnp.int32)
    return pl.pallas_call(
        _lin_body,
        grid=((VOCAB + HB - 1) // HB,),
        in_specs=[spec_i, spec_i, spec_i],
        out_specs=[spec_o, spec_o, spec_o],
        out_shape=[out_t, out_t, out_t],
    )(h0t, h1t, h2t)


def kernel(x, table0, table1, weights, h0, h1, h2):
    t0w, t1w = _tab_linearize(jnp.transpose(table0, (1, 2, 0)),
                              jnp.transpose(table1, (1, 2, 0)))
    t0 = t0w.reshape(ROWS * N_CHUNKS, CHUNK)
    t1 = t1w.reshape(ROWS * N_CHUNKS, CHUNK)
    w = weights.reshape(ROWS * N_CHUNKS * 2)
    h0f, h1f, h2f = (h.reshape(PTILES * N_CHUNKS * 128)
                     for h in _linearize(jnp.transpose(h0), jnp.transpose(h1),
                                         jnp.transpose(h2)))
    return _call(x, t0, t1, w, h0f, h1f, h2f)


# speedup vs baseline: 11.0867x; 2.4158x over previous
import jax
import jax.numpy as jnp
from jax import lax
from jax.experimental import pallas as pl
from jax.experimental.pallas import tpu as pltpu
from jax.experimental.pallas import tpu_sc as plsc

VOCAB = 1000000
ROWS = 65536
N_CHUNKS = 8
CHUNK = 16
B = 16384

NC = 2
NS = 16
NW = NC * NS
TPW = B // NW
RPW = TPW * N_CHUNKS
SUB = 4
CPS = N_CHUNKS // SUB
RPS = TPW * CPS
IDX_W = 128
KPW = RPW // IDX_W
K_SUB = RPS // IDX_W


def _body(x_hbm, t0_hbm, t1_hbm, w_hbm, h0_hbm, h1_hbm, h2_hbm, out_hbm,
          x_v, xe_v, g0_v, g1_v, g2_v, w0_v, w1_v, p0_v, p1_v, out_v, sem):
    c = lax.axis_index("c")
    s = lax.axis_index("s")
    wid = s * NC + c
    tok_base = wid * TPW

    pltpu.sync_copy(x_hbm.at[pl.ds(tok_base, TPW)], x_v)

    def xe_body(i, _):
        cc = lax.shift_right_logical(i, 5)
        bo = jnp.bitwise_and(i, 31) * 16
        xv = x_v[pl.ds(bo, 16)]
        xe_v[pl.ds(i * 16, 16)] = (
            lax.shift_left(lax.shift_right_logical(xv, 7), 10)
            + cc * 128 + jnp.bitwise_and(xv, 127))
        return 0

    lax.fori_loop(0, RPW // 16, xe_body, 0)

    cp0 = pltpu.async_copy(h0_hbm.at[xe_v], g0_v, sem)
    cp1 = pltpu.async_copy(h1_hbm.at[xe_v], g1_v, sem)
    cp2 = pltpu.async_copy(h2_hbm.at[xe_v], g2_v, sem)
    cp0.wait()
    cp1.wait()
    cp2.wait()

    def idx_body(i, _):
        cc = lax.shift_right_logical(i, 5)
        sl = pl.ds(i * 16, 16)
        g0_v[sl] = g0_v[sl] * N_CHUNKS + cc
        g1_v[sl] = g1_v[sl] * N_CHUNKS + cc
        h2v = g2_v[sl]
        base = (cc * (2 * ROWS)
                + lax.shift_left(lax.shift_right_logical(h2v, 7), 8)
                + jnp.bitwise_and(h2v, 127))
        xe_v[sl] = base
        g2_v[sl] = base + 128
        return 0

    lax.fori_loop(0, RPW // 16, idx_body, 0)

    cpw0 = pltpu.async_copy(w_hbm.at[xe_v], w0_v, sem)
    cpw1 = pltpu.async_copy(w_hbm.at[g2_v], w1_v, sem)
    cpw0.wait()
    cpw1.wait()

    for sb in range(SUB):
        sl = pl.ds(sb * RPS, RPS)
        cpt0 = pltpu.async_copy(t0_hbm.at[g0_v.at[sl]], p0_v, sem)
        cpt1 = pltpu.async_copy(t1_hbm.at[g1_v.at[sl]], p1_v, sem)
        cpt0.wait()
        cpt1.wait()

        def row_body(m, _):
            q = sb * RPS + m
            qv = jnp.full((16,), q, jnp.int32)
            w0 = plsc.load_gather(w0_v, [qv])
            w1 = plsc.load_gather(w1_v, [qv])
            cc = sb * CPS + lax.shift_right_logical(m, 9)
            b = jnp.bitwise_and(m, TPW - 1)
            out_v[b, pl.ds(cc * CHUNK, CHUNK)] = (
                p0_v[m, :] * w0 + p1_v[m, :] * w1)
            return 0

        lax.fori_loop(0, RPS, row_body, 0)

    pltpu.sync_copy(out_v, out_hbm.at[pl.ds(tok_base, TPW), :])


@jax.jit
def _call(x, t0, t1, w, h0f, h1f, h2f):
    mesh = plsc.VectorSubcoreMesh(core_axis_name="c", subcore_axis_name="s")
    run = pl.kernel(
        _body,
        out_type=jax.ShapeDtypeStruct((B, N_CHUNKS * CHUNK), jnp.float32),
        mesh=mesh,
        compiler_params=pltpu.CompilerParams(use_tc_tiling_on_sc=False,
                                             needs_layout_passes=False),
        scratch_types=[
            pltpu.VMEM((TPW,), jnp.int32),
            pltpu.VMEM((RPW,), jnp.int32),
            pltpu.VMEM((RPW,), jnp.int32),
            pltpu.VMEM((RPW,), jnp.int32),
            pltpu.VMEM((RPW,), jnp.int32),
            pltpu.VMEM((RPW,), jnp.float32),
            pltpu.VMEM((RPW,), jnp.float32),
            pltpu.VMEM((RPS, CHUNK), jnp.float32),
            pltpu.VMEM((RPS, CHUNK), jnp.float32),
            pltpu.VMEM((TPW, 128), jnp.float32),
            pltpu.SemaphoreType.DMA,
        ],
    )
    return run(x, t0, t1, w, h0f, h1f, h2f)


HB = 65536
TILES = (VOCAB + 127) // 128
PTILES = 7816
HPAD = PTILES * 128


def _lin_body(i0_ref, i1_ref, i2_ref, o0_ref, o1_ref, o2_ref):
    for i_ref, o_ref in ((i0_ref, o0_ref), (i1_ref, o1_ref),
                         (i2_ref, o2_ref)):
        o_ref[...] = jnp.transpose(
            i_ref[...].reshape(N_CHUNKS, HB // 128, 128), (1, 0, 2))


def _w_body(w_ref, o_ref):
    t = w_ref[...].reshape(N_CHUNKS, 2, ROWS // 128, 128)
    o_ref[...] = jnp.transpose(t, (0, 2, 1, 3)).reshape(
        N_CHUNKS, ROWS // 64, 128)


def _w_linearize(wp):
    return pl.pallas_call(
        _w_body,
        in_specs=[pl.BlockSpec((N_CHUNKS, 2, ROWS), lambda: (0, 0, 0))],
        out_specs=pl.BlockSpec((N_CHUNKS, ROWS // 64, 128),
                               lambda: (0, 0, 0)),
        out_shape=jax.ShapeDtypeStruct((N_CHUNKS, ROWS // 64, 128),
                                       jnp.float32),
    )(wp)


WT = 4096


def _tab_body(t0_ref, t1_ref, o0_ref, o1_ref):
    o0_ref[...] = jnp.transpose(t0_ref[...].reshape(128, WT))
    o1_ref[...] = jnp.transpose(t1_ref[...].reshape(128, WT))


def _tab_linearize(t0p, t1p):
    spec_i = pl.BlockSpec((N_CHUNKS, CHUNK, WT), lambda rb: (0, 0, rb))
    spec_o = pl.BlockSpec((WT, 128), lambda rb: (rb, 0))
    out_t = jax.ShapeDtypeStruct((ROWS, 128), jnp.float32)
    return pl.pallas_call(
        _tab_body,
        grid=(ROWS // WT,),
        in_specs=[spec_i, spec_i],
        out_specs=[spec_o, spec_o],
        out_shape=[out_t, out_t],
    )(t0p, t1p)


def _linearize(h0t, h1t, h2t):
    spec_i = pl.BlockSpec((N_CHUNKS, HB), lambda xb: (0, xb))
    spec_o = pl.BlockSpec((HB // 128, N_CHUNKS, 128), lambda xb: (xb, 0, 0))
    out_t = jax.ShapeDtypeStruct((PTILES, N_CHUNKS, 128), jnp.int32)
    return pl.pallas_call(
        _lin_body,
        grid=((VOCAB + HB - 1) // HB,),
        in_specs=[spec_i, spec_i, spec_i],
        out_specs=[spec_o, spec_o, spec_o],
        out_shape=[out_t, out_t, out_t],
    )(h0t, h1t, h2t)


def kernel(x, table0, table1, weights, h0, h1, h2):
    t0w, t1w = _tab_linearize(jnp.transpose(table0, (1, 2, 0)),
                              jnp.transpose(table1, (1, 2, 0)))
    t0 = t0w.reshape(ROWS * N_CHUNKS, CHUNK)
    t1 = t1w.reshape(ROWS * N_CHUNKS, CHUNK)
    w = _w_linearize(jnp.transpose(weights, (1, 2, 0))).reshape(
        ROWS * N_CHUNKS * 2)
    h0f, h1f, h2f = (h.reshape(PTILES * N_CHUNKS * 128)
                     for h in _linearize(jnp.transpose(h0), jnp.transpose(h1),
                                         jnp.transpose(h2)))
    return _call(x, t0, t1, w, h0f, h1f, h2f)


# speedup vs baseline: 11.6928x; 1.0547x over previous
import jax
import jax.numpy as jnp
from jax import lax
from jax.experimental import pallas as pl
from jax.experimental.pallas import tpu as pltpu
from jax.experimental.pallas import tpu_sc as plsc

VOCAB = 1000000
ROWS = 65536
N_CHUNKS = 8
CHUNK = 16
B = 16384

NC = 2
NS = 16
NW = NC * NS
TPW = B // NW
RPW = TPW * N_CHUNKS
RPS = TPW


def _body(x_hbm, t0_hbm, t1_hbm, w_hbm, h0_hbm, h1_hbm, h2_hbm, out_hbm,
          x_v, xe_v, g0_v, g1_v, g2_v, w0_v, w1_v, p0a, p1a, p0b, p1b,
          out_v, sem, sem_b):
    c = lax.axis_index("c")
    s = lax.axis_index("s")
    wid = s * NC + c
    tok_base = wid * TPW

    pltpu.sync_copy(x_hbm.at[pl.ds(tok_base, TPW)], x_v)

    def xe_body(i, _):
        cc = lax.shift_right_logical(i, 5)
        bo = jnp.bitwise_and(i, 31) * 16
        xv = x_v[pl.ds(bo, 16)]
        xe_v[pl.ds(i * 16, 16)] = (
            lax.shift_left(lax.shift_right_logical(xv, 7), 10)
            + cc * 128 + jnp.bitwise_and(xv, 127))
        return 0

    lax.fori_loop(0, RPW // 16, xe_body, 0)

    cp0 = pltpu.async_copy(h0_hbm.at[xe_v], g0_v, sem)
    cp1 = pltpu.async_copy(h1_hbm.at[xe_v], g1_v, sem)
    cp2 = pltpu.async_copy(h2_hbm.at[xe_v], g2_v, sem)
    cp0.wait()
    cp1.wait()
    cp2.wait()

    def idx_body(i, _):
        cc = lax.shift_right_logical(i, 5)
        sl = pl.ds(i * 16, 16)
        g0_v[sl] = g0_v[sl] * N_CHUNKS + cc
        g1_v[sl] = g1_v[sl] * N_CHUNKS + cc
        h2v = g2_v[sl]
        base = (cc * (2 * ROWS)
                + lax.shift_left(lax.shift_right_logical(h2v, 7), 8)
                + jnp.bitwise_and(h2v, 127))
        xe_v[sl] = base
        g2_v[sl] = base + 128
        return 0

    lax.fori_loop(0, RPW // 16, idx_body, 0)

    cpw0 = pltpu.async_copy(w_hbm.at[xe_v], w0_v, sem)
    cpw1 = pltpu.async_copy(w_hbm.at[g2_v], w1_v, sem)
    cpw0.wait()
    cpw1.wait()

    bufs = ((p0a, p1a), (p0b, p1b))
    sems = (sem, sem_b)

    def fire(sb):
        bb = sb & 1
        sl = pl.ds(sb * RPS, RPS)
        return (pltpu.async_copy(t0_hbm.at[g0_v.at[sl]], bufs[bb][0],
                                 sems[bb]),
                pltpu.async_copy(t1_hbm.at[g1_v.at[sl]], bufs[bb][1],
                                 sems[bb]))

    cur = fire(0)
    for sb in range(N_CHUNKS):
        nxt = fire(sb + 1) if sb + 1 < N_CHUNKS else None
        cur[0].wait()
        cur[1].wait()
        p0_v, p1_v = bufs[sb & 1]

        def row_body(m, _):
            qv = jnp.full((16,), sb * RPS + m, jnp.int32)
            w0 = plsc.load_gather(w0_v, [qv])
            w1 = plsc.load_gather(w1_v, [qv])
            out_v[m, pl.ds(sb * CHUNK, CHUNK)] = (
                p0_v[m, :] * w0 + p1_v[m, :] * w1)
            return 0

        lax.fori_loop(0, RPS, row_body, 0)
        cur = nxt

    pltpu.sync_copy(out_v, out_hbm.at[pl.ds(tok_base, TPW), :])


@jax.jit
def _call(x, t0, t1, w, h0f, h1f, h2f):
    mesh = plsc.VectorSubcoreMesh(core_axis_name="c", subcore_axis_name="s")
    run = pl.kernel(
        _body,
        out_type=jax.ShapeDtypeStruct((B, N_CHUNKS * CHUNK), jnp.float32),
        mesh=mesh,
        compiler_params=pltpu.CompilerParams(use_tc_tiling_on_sc=False,
                                             needs_layout_passes=False),
        scratch_types=[
            pltpu.VMEM((TPW,), jnp.int32),
            pltpu.VMEM((RPW,), jnp.int32),
            pltpu.VMEM((RPW,), jnp.int32),
            pltpu.VMEM((RPW,), jnp.int32),
            pltpu.VMEM((RPW,), jnp.int32),
            pltpu.VMEM((RPW,), jnp.float32),
            pltpu.VMEM((RPW,), jnp.float32),
            pltpu.VMEM((RPS, CHUNK), jnp.float32),
            pltpu.VMEM((RPS, CHUNK), jnp.float32),
            pltpu.VMEM((RPS, CHUNK), jnp.float32),
            pltpu.VMEM((RPS, CHUNK), jnp.float32),
            pltpu.VMEM((TPW, 128), jnp.float32),
            pltpu.SemaphoreType.DMA,
            pltpu.SemaphoreType.DMA,
        ],
    )
    return run(x, t0, t1, w, h0f, h1f, h2f)


HB = 65536
TILES = (VOCAB + 127) // 128
PTILES = 7816
HPAD = PTILES * 128


def _lin_body(i0_ref, i1_ref, i2_ref, o0_ref, o1_ref, o2_ref):
    for i_ref, o_ref in ((i0_ref, o0_ref), (i1_ref, o1_ref),
                         (i2_ref, o2_ref)):
        o_ref[...] = jnp.transpose(
            i_ref[...].reshape(N_CHUNKS, HB // 128, 128), (1, 0, 2))


def _w_body(w_ref, o_ref):
    t = w_ref[...].reshape(N_CHUNKS, 2, ROWS // 128, 128)
    o_ref[...] = jnp.transpose(t, (0, 2, 1, 3)).reshape(
        N_CHUNKS, ROWS // 64, 128)


def _w_linearize(wp):
    return pl.pallas_call(
        _w_body,
        in_specs=[pl.BlockSpec((N_CHUNKS, 2, ROWS), lambda: (0, 0, 0))],
        out_specs=pl.BlockSpec((N_CHUNKS, ROWS // 64, 128),
                               lambda: (0, 0, 0)),
        out_shape=jax.ShapeDtypeStruct((N_CHUNKS, ROWS // 64, 128),
                                       jnp.float32),
    )(wp)


WT = 4096


def _tab_body(t0_ref, t1_ref, o0_ref, o1_ref):
    o0_ref[...] = jnp.transpose(t0_ref[...].reshape(128, WT))
    o1_ref[...] = jnp.transpose(t1_ref[...].reshape(128, WT))


def _tab_linearize(t0p, t1p):
    spec_i = pl.BlockSpec((N_CHUNKS, CHUNK, WT), lambda rb: (0, 0, rb))
    spec_o = pl.BlockSpec((WT, 128), lambda rb: (rb, 0))
    out_t = jax.ShapeDtypeStruct((ROWS, 128), jnp.float32)
    return pl.pallas_call(
        _tab_body,
        grid=(ROWS // WT,),
        in_specs=[spec_i, spec_i],
        out_specs=[spec_o, spec_o],
        out_shape=[out_t, out_t],
    )(t0p, t1p)


def _linearize(h0t, h1t, h2t):
    spec_i = pl.BlockSpec((N_CHUNKS, HB), lambda xb: (0, xb))
    spec_o = pl.BlockSpec((HB // 128, N_CHUNKS, 128), lambda xb: (xb, 0, 0))
    out_t = jax.ShapeDtypeStruct((PTILES, N_CHUNKS, 128), jnp.int32)
    return pl.pallas_call(
        _lin_body,
        grid=((VOCAB + HB - 1) // HB,),
        in_specs=[spec_i, spec_i, spec_i],
        out_specs=[spec_o, spec_o, spec_o],
        out_shape=[out_t, out_t, out_t],
    )(h0t, h1t, h2t)


def kernel(x, table0, table1, weights, h0, h1, h2):
    t0w, t1w = _tab_linearize(jnp.transpose(table0, (1, 2, 0)),
                              jnp.transpose(table1, (1, 2, 0)))
    t0 = t0w.reshape(ROWS * N_CHUNKS, CHUNK)
    t1 = t1w.reshape(ROWS * N_CHUNKS, CHUNK)
    w = _w_linearize(jnp.transpose(weights, (1, 2, 0))).reshape(
        ROWS * N_CHUNKS * 2)
    h0f, h1f, h2f = (h.reshape(PTILES * N_CHUNKS * 128)
                     for h in _linearize(jnp.transpose(h0), jnp.transpose(h1),
                                         jnp.transpose(h2)))
    return _call(x, t0, t1, w, h0f, h1f, h2f)
